# baseline (speedup 1.0000x reference)
import jax
import jax.numpy as jnp
from jax.experimental import pallas as pl
from jax.experimental.pallas import tpu as pltpu

_H = 768
_H2 = 384
_H4 = 192
_NEXP = 8


def _fused_kernel(x_ref, w1c_ref, b1c_ref, w2_ref, b2_ref, w3_ref, b3_ref,
                  wv2_ref, bv2_ref, logits_ref, values_ref):
    x = x_ref[...]
    h1 = jnp.dot(x, w1c_ref[...], preferred_element_type=jnp.float32)
    h1 = jnp.maximum(h1 + b1c_ref[...], 0.0)
    h = h1[:, :_H2]
    v = h1[:, _H2:]
    h2 = jnp.dot(h, w2_ref[...], preferred_element_type=jnp.float32)
    h2 = jnp.maximum(h2 + b2_ref[...], 0.0)
    logits_ref[...] = (
        jnp.dot(h2, w3_ref[...], preferred_element_type=jnp.float32)
        + b3_ref[...]
    )
    values_ref[...] = (
        jnp.dot(v, wv2_ref[...], preferred_element_type=jnp.float32)
        + bv2_ref[...]
    )


def kernel(hidden_states, W1, b1, W2, b2, W3, b3, Wv1, bv1, Wv2, bv2):
    B, S, H = hidden_states.shape
    n_tok = B * S
    flat = hidden_states.reshape(n_tok, H)

    w1c = jnp.concatenate([W1, Wv1], axis=1)
    b1c = jnp.concatenate([b1, bv1])[None, :]

    block = 1024
    grid = (n_tok // block,)

    logits, values = pl.pallas_call(
        _fused_kernel,
        grid=grid,
        in_specs=[
            pl.BlockSpec((block, H), lambda i: (i, 0)),
            pl.BlockSpec((H, 2 * _H2), lambda i: (0, 0)),
            pl.BlockSpec((1, 2 * _H2), lambda i: (0, 0)),
            pl.BlockSpec((_H2, _H4), lambda i: (0, 0)),
            pl.BlockSpec((1, _H4), lambda i: (0, 0)),
            pl.BlockSpec((_H4, _NEXP), lambda i: (0, 0)),
            pl.BlockSpec((1, _NEXP), lambda i: (0, 0)),
            pl.BlockSpec((_H2, 1), lambda i: (0, 0)),
            pl.BlockSpec((1, 1), lambda i: (0, 0)),
        ],
        out_specs=[
            pl.BlockSpec((block, _NEXP), lambda i: (i, 0)),
            pl.BlockSpec((block, 1), lambda i: (i, 0)),
        ],
        out_shape=[
            jax.ShapeDtypeStruct((n_tok, _NEXP), jnp.float32),
            jax.ShapeDtypeStruct((n_tok, 1), jnp.float32),
        ],
    )(flat, w1c, b1c, W2, b2[None, :], W3, b3[None, :], Wv2, bv2[None, :])

    return (logits.reshape(B, S, _NEXP), values.reshape(B, S, 1))


# speedup vs baseline: 1.0024x; 1.0024x over previous
import jax
import jax.numpy as jnp
from jax.experimental import pallas as pl
from jax.experimental.pallas import tpu as pltpu

_H = 768
_H2 = 384
_H4 = 192
_NEXP = 8


def _fused_kernel(x_ref, w1c_ref, b1c_ref, w2_ref, b2_ref, w3_ref, b3_ref,
                  wv2_ref, bv2_ref, logits_ref, values_ref):
    x = x_ref[...]
    h1 = jnp.dot(x, w1c_ref[...], preferred_element_type=jnp.float32)
    h1 = jnp.maximum(h1 + b1c_ref[...], 0.0)
    h = h1[:, :_H2]
    v = h1[:, _H2:]
    h2 = jnp.dot(h, w2_ref[...], preferred_element_type=jnp.float32)
    h2 = jnp.maximum(h2 + b2_ref[...], 0.0)
    logits_ref[...] = (
        jnp.dot(h2, w3_ref[...], preferred_element_type=jnp.float32)
        + b3_ref[...]
    )
    values_ref[...] = (
        jnp.dot(v, wv2_ref[...], preferred_element_type=jnp.float32)
        + bv2_ref[...]
    )


def kernel(hidden_states, W1, b1, W2, b2, W3, b3, Wv1, bv1, Wv2, bv2):
    B, S, H = hidden_states.shape
    n_tok = B * S
    flat = hidden_states.reshape(n_tok, H)

    w1c = jnp.concatenate([W1, Wv1], axis=1)
    b1c = jnp.concatenate([b1, bv1])[None, :]

    block = 1024
    grid = (n_tok // block,)

    logits, values = pl.pallas_call(
        _fused_kernel,
        grid=grid,
        in_specs=[
            pl.BlockSpec((block, H), lambda i: (i, 0)),
            pl.BlockSpec((H, 2 * _H2), lambda i: (0, 0)),
            pl.BlockSpec((1, 2 * _H2), lambda i: (0, 0)),
            pl.BlockSpec((_H2, _H4), lambda i: (0, 0)),
            pl.BlockSpec((1, _H4), lambda i: (0, 0)),
            pl.BlockSpec((_H4, _NEXP), lambda i: (0, 0)),
            pl.BlockSpec((1, _NEXP), lambda i: (0, 0)),
            pl.BlockSpec((_H2, 1), lambda i: (0, 0)),
            pl.BlockSpec((1, 1), lambda i: (0, 0)),
        ],
        out_specs=[
            pl.BlockSpec((block, _NEXP), lambda i: (i, 0)),
            pl.BlockSpec((block, 1), lambda i: (i, 0)),
        ],
        out_shape=[
            jax.ShapeDtypeStruct((n_tok, _NEXP), jnp.float32),
            jax.ShapeDtypeStruct((n_tok, 1), jnp.float32),
        ],
        compiler_params=pltpu.CompilerParams(
            dimension_semantics=("parallel",),
        ),
    )(flat, w1c, b1c, W2, b2[None, :], W3, b3[None, :], Wv2, bv2[None, :])

    return (logits.reshape(B, S, _NEXP), values.reshape(B, S, 1))


# speedup vs baseline: 1.0714x; 1.0688x over previous
import jax
import jax.numpy as jnp
from jax.experimental import pallas as pl
from jax.experimental.pallas import tpu as pltpu

_H = 768
_H2 = 384
_H4 = 192
_NEXP = 8


def _fused_kernel(x_ref, w1c_ref, b1c_ref, w2_ref, b2_ref, w3_ref, b3_ref,
                  wv2_ref, bv2_ref, logits_ref, values_ref):
    x = x_ref[...]
    h1 = jnp.dot(x, w1c_ref[...], preferred_element_type=jnp.float32)
    h1 = jnp.maximum(h1 + b1c_ref[...], 0.0)
    h = h1[:, :_H2]
    v = h1[:, _H2:]
    h2 = jnp.dot(h, w2_ref[...], preferred_element_type=jnp.float32)
    h2 = jnp.maximum(h2 + b2_ref[...], 0.0)
    logits_ref[...] = (
        jnp.dot(h2, w3_ref[...], preferred_element_type=jnp.float32)
        + b3_ref[...]
    )
    values_ref[...] = (
        jnp.dot(v, wv2_ref[...], preferred_element_type=jnp.float32)
        + bv2_ref[...]
    )


def kernel(hidden_states, W1, b1, W2, b2, W3, b3, Wv1, bv1, Wv2, bv2):
    B, S, H = hidden_states.shape
    n_tok = B * S
    flat = hidden_states.reshape(n_tok, H)

    w1c = jnp.concatenate([W1, Wv1], axis=1)
    b1c = jnp.concatenate([b1, bv1])[None, :]

    block = 2048
    grid = (n_tok // block,)

    logits, values = pl.pallas_call(
        _fused_kernel,
        grid=grid,
        in_specs=[
            pl.BlockSpec((block, H), lambda i: (i, 0)),
            pl.BlockSpec((H, 2 * _H2), lambda i: (0, 0)),
            pl.BlockSpec((1, 2 * _H2), lambda i: (0, 0)),
            pl.BlockSpec((_H2, _H4), lambda i: (0, 0)),
            pl.BlockSpec((1, _H4), lambda i: (0, 0)),
            pl.BlockSpec((_H4, _NEXP), lambda i: (0, 0)),
            pl.BlockSpec((1, _NEXP), lambda i: (0, 0)),
            pl.BlockSpec((_H2, 1), lambda i: (0, 0)),
            pl.BlockSpec((1, 1), lambda i: (0, 0)),
        ],
        out_specs=[
            pl.BlockSpec((block, _NEXP), lambda i: (i, 0)),
            pl.BlockSpec((block, 1), lambda i: (i, 0)),
        ],
        out_shape=[
            jax.ShapeDtypeStruct((n_tok, _NEXP), jnp.float32),
            jax.ShapeDtypeStruct((n_tok, 1), jnp.float32),
        ],
        compiler_params=pltpu.CompilerParams(
            dimension_semantics=("parallel",),
        ),
    )(flat, w1c, b1c, W2, b2[None, :], W3, b3[None, :], Wv2, bv2[None, :])

    return (logits.reshape(B, S, _NEXP), values.reshape(B, S, 1))
